# baseline (speedup 1.0000x reference)
import jax
import jax.numpy as jnp
from jax.experimental import pallas as pl
from jax.experimental.pallas import tpu as pltpu


def _q2c_kernel(h_ref, s_ref, o_ref):
    s = s_ref[0]
    h = h_ref[0]
    z1 = jnp.max(s, axis=-1, keepdims=True)
    m = jnp.max(z1, axis=0, keepdims=True)
    e = jnp.exp(z1 - m)
    denom = jnp.sum(e, axis=0, keepdims=True)
    p = jnp.sum(e * h, axis=0, keepdims=True)
    pooled = p / denom
    o_ref[0] = jnp.broadcast_to(pooled, h.shape)


def kernel(h, s):
    B, T, D = h.shape
    J = s.shape[-1]
    return pl.pallas_call(
        _q2c_kernel,
        grid=(B,),
        in_specs=[
            pl.BlockSpec((1, T, D), lambda b: (b, 0, 0)),
            pl.BlockSpec((1, T, J), lambda b: (b, 0, 0)),
        ],
        out_specs=pl.BlockSpec((1, T, D), lambda b: (b, 0, 0)),
        out_shape=jax.ShapeDtypeStruct(h.shape, h.dtype),
        compiler_params=pltpu.CompilerParams(
            dimension_semantics=("parallel",),
        ),
    )(h, s)
